# baseline (speedup 1.0000x reference)
import functools

import numpy as np

import jax
import jax.numpy as jnp
from jax import lax
from jax.experimental import pallas as pl
from jax.experimental.pallas import tpu as pltpu
from jax.experimental.pallas import tpu_sc as plsc

_HEADS = 2
_IMG = 224
_EMB = 192
_DH = _EMB // _HEADS
_CH = 384
_P = _IMG * _IMG
_EPS = 1e-5
_NUM_OBJ = _IMG // 2
_GROUPS = _HEADS * _IMG
_PAD = 128


def _conv1_body(x_ref, w_ref, b_ref, o0_ref, o1_ref):
    t = lax.dot_general(x_ref[...], w_ref[...], (((0,), (1,)), ((), ())),
                        preferred_element_type=jnp.float32)
    t = jnp.maximum(t + b_ref[...], 0.0)
    z = jnp.zeros((t.shape[0], _PAD - _DH), jnp.float32)
    o0_ref[...] = jnp.concatenate([t[:, :_DH], z], axis=1)
    o1_ref[...] = jnp.concatenate([t[:, _DH:], z], axis=1)


def _conv1(x2d, w, b, tp=1792):
    grid = _P // tp
    return pl.pallas_call(
        _conv1_body,
        grid=(grid,),
        in_specs=[
            pl.BlockSpec((_CH, tp), lambda i: (0, i)),
            pl.BlockSpec((_EMB, _CH), lambda i: (0, 0)),
            pl.BlockSpec((1, _EMB), lambda i: (0, 0)),
        ],
        out_specs=[
            pl.BlockSpec((tp, _PAD), lambda i: (i, 0)),
            pl.BlockSpec((tp, _PAD), lambda i: (i, 0)),
        ],
        out_shape=[
            jax.ShapeDtypeStruct((_P, _PAD), jnp.float32),
            jax.ShapeDtypeStruct((_P, _PAD), jnp.float32),
        ],
    )(x2d, w, b)


_AG = 8
_RUNS = [
    [(0, 0, 96), (1, 0, 96), (2, 0, 32)],
    [(2, 32, 64), (3, 0, 96), (4, 0, 64)],
    [(4, 64, 32), (5, 0, 96), (6, 0, 96)],
]


def _attn_body(g_ref, wq_ref, wk_ref, wv_ref, o_ref):
    for b in range(_AG):
        g = g_ref[b][:, :_DH]
        q = lax.dot_general(g, wq_ref[...], (((1,), (1,)), ((), ())),
                            preferred_element_type=jnp.float32)
        k = lax.dot_general(g, wk_ref[...], (((1,), (1,)), ((), ())),
                            preferred_element_type=jnp.float32)
        v = lax.dot_general(g, wv_ref[...], (((1,), (1,)), ((), ())),
                            preferred_element_type=jnp.float32)
        dots = lax.dot_general(q, k, (((1,), (1,)), ((), ())),
                               preferred_element_type=jnp.float32)
        dots = dots * (_DH ** -0.5)
        m = jnp.max(dots, axis=-1, keepdims=True)
        e = jnp.exp(dots - m)
        s = jnp.sum(e, axis=-1, keepdims=True)
        attn = e / s
        o = lax.dot_general(attn, v, (((1,), (0,)), ((), ())),
                            preferred_element_type=jnp.float32)
        o3 = o.reshape(32, 7, _DH)
        pieces = [jnp.concatenate(
            [lax.transpose(o3[:, qq, c0:c0 + ln], (1, 0))
             for (qq, c0, ln) in _RUNS[t]], axis=0)
            for t in range(3)]
        o_ref[b] = jnp.concatenate(pieces, axis=1)


def _attention(g, wq, wk, wv):
    return pl.pallas_call(
        _attn_body,
        grid=(_GROUPS // _AG,),
        in_specs=[
            pl.BlockSpec((_AG, _IMG, _PAD), lambda i: (i, 0, 0)),
            pl.BlockSpec((_DH, _DH), lambda i: (0, 0)),
            pl.BlockSpec((_DH, _DH), lambda i: (0, 0)),
            pl.BlockSpec((_DH, _DH), lambda i: (0, 0)),
        ],
        out_specs=pl.BlockSpec((_AG, _IMG, _DH), lambda i: (i, 0, 0)),
        out_shape=jax.ShapeDtypeStruct((_GROUPS, _IMG, _DH), jnp.float32),
    )(g, wq, wk, wv)


def _final_body(n0_ref, n1_ref, h0_ref, h1_ref, wo_ref, bo_ref, w2a_ref,
                w2b_ref, b2_ref, o_ref):
    n = jnp.concatenate([n0_ref[:, :_DH], n1_ref[:, :_DH]], axis=1)
    a = lax.dot_general(n, wo_ref[...], (((1,), (1,)), ((), ())),
                        preferred_element_type=jnp.float32)
    a = jnp.maximum(a + bo_ref[...], 0.0)
    y = lax.dot_general(w2a_ref[...], a, (((1,), (1,)), ((), ())),
                        preferred_element_type=jnp.float32)
    hcat = jnp.concatenate([h0_ref[:, :_DH], h1_ref[:, :_DH]], axis=1)
    y = y + lax.dot_general(w2b_ref[...], hcat, (((1,), (1,)), ((), ())),
                            preferred_element_type=jnp.float32)
    o_ref[...] = jnp.maximum(y + b2_ref[...], 0.0)


def _final(new, h0, h1, wo, bo, w2a, w2b, b2, tp=1792):
    grid = _P // tp
    off = _P // tp
    return pl.pallas_call(
        _final_body,
        grid=(grid,),
        in_specs=[
            pl.BlockSpec((tp, _PAD), lambda i: (i, 0)),
            pl.BlockSpec((tp, _PAD), lambda i: (off + i, 0)),
            pl.BlockSpec((tp, _PAD), lambda i: (i, 0)),
            pl.BlockSpec((tp, _PAD), lambda i: (i, 0)),
            pl.BlockSpec((_EMB, _EMB), lambda i: (0, 0)),
            pl.BlockSpec((1, _EMB), lambda i: (0, 0)),
            pl.BlockSpec((_EMB, _EMB), lambda i: (0, 0)),
            pl.BlockSpec((_EMB, _EMB), lambda i: (0, 0)),
            pl.BlockSpec((_EMB, 1), lambda i: (0, 0)),
        ],
        out_specs=pl.BlockSpec((_EMB, tp), lambda i: (0, i)),
        out_shape=jax.ShapeDtypeStruct((_EMB, _P), jnp.float32),
    )(new, new, h0, h1, wo, bo, w2a, w2b, b2)


_NW = 32
_BPW = _P // _NW
_GC = 392


def _sc_gather2(t0, t1, i0, i1):
    mesh = plsc.VectorSubcoreMesh(core_axis_name="c", subcore_axis_name="s")

    @functools.partial(
        pl.kernel, mesh=mesh,
        out_type=jax.ShapeDtypeStruct((2 * _P, _PAD), jnp.float32),
        scratch_types=[
            pltpu.VMEM((_GC,), jnp.int32),
            pltpu.VMEM((_GC, _PAD), jnp.float32),
            pltpu.SemaphoreType.DMA,
        ],
    )
    def k(t0_hbm, t1_hbm, i0_hbm, i1_hbm, out_hbm, idx_v, rows_v, sem):
        wid = lax.axis_index("s") * 2 + lax.axis_index("c")
        for h in range(_HEADS):
            tab = t0_hbm if h == 0 else t1_hbm
            idx = i0_hbm if h == 0 else i1_hbm
            for c in range(_BPW // _GC):
                off = wid * _BPW + c * _GC
                pltpu.sync_copy(idx.at[pl.ds(off, _GC)], idx_v)
                pltpu.async_copy(tab.at[idx_v], rows_v, sem).wait()
                pltpu.sync_copy(rows_v, out_hbm.at[pl.ds(h * _P + off, _GC)])

    return k(t0, t1, i0, i1)


def _build_pix(prop, rand_inds):
    p = jnp.reshape(prop, (-1,))
    obj_mask = p > 0
    bg_mask = p <= 0
    obj_cnt = jnp.sum(obj_mask)
    bg_cnt = jnp.sum(bg_mask)
    obj_full = jnp.nonzero(obj_mask, size=_P, fill_value=0)[0]
    bg_full = jnp.nonzero(bg_mask, size=_P, fill_value=0)[0]
    ar = jnp.arange(_P)
    obj = jnp.where(obj_cnt == 0, ar, obj_full)
    bg = jnp.where(bg_cnt == 0, ar, bg_full)
    obj_sz = jnp.where(obj_cnt == 0, _P, obj_cnt)
    bg_sz = jnp.where(bg_cnt == 0, _P, bg_cnt)
    pix = []
    for h in range(_HEADS):
        op = obj[rand_inds[h, :_NUM_OBJ] % obj_sz]
        bp = bg[rand_inds[h, _NUM_OBJ:] % bg_sz]
        pix.append(jnp.concatenate([op, bp], axis=0).reshape(-1))
    return pix


def kernel(x, prop, conv1_w, bn1_gamma, bn1_beta, Wq, Wkv, Wout, bout,
           conv2_w, bn2_gamma, bn2_beta, rand_inds):
    inv = 1.0 / jnp.sqrt(jnp.float32(1.0 + _EPS))
    sig = np.array([3 * (m % 32) + m // 32 for m in range(_DH)])
    sig2 = np.concatenate([sig, _DH + sig])
    w1 = conv1_w.reshape(_EMB, _CH) * (bn1_gamma * inv)[:, None]
    w1 = w1[sig2]
    b1 = bn1_beta[sig2].reshape(1, _EMB)
    w2 = conv2_w.reshape(_EMB, 2 * _EMB) * (bn2_gamma * inv)[:, None]
    b2 = bn2_beta.reshape(_EMB, 1)
    wq = Wq[:, sig]
    wk, wv = Wkv[:_DH, sig], Wkv[_DH:, sig]
    wo = Wout[:, sig2]

    x2d = x.reshape(_CH, _P)
    h0, h1 = _conv1(x2d, w1, b1)
    bases = (h0, h1)

    pix = _build_pix(prop, rand_inds)

    g = _sc_gather2(h0, h1, pix[0], pix[1])
    g = g.reshape(_GROUPS, _IMG, _PAD)

    out = _attention(g, wq, wk, wv)

    ar = jnp.arange(_P, dtype=jnp.int32)
    winner = jnp.full((2 * _P,), -1, jnp.int32).at[
        jnp.concatenate([pix[0], _P + pix[1]])].max(
        jnp.concatenate([ar, ar]))
    tables, idx2 = [], []
    for h in range(_HEADS):
        vals = out[h * _IMG:(h + 1) * _IMG].reshape(_P, _DH)
        vals = jnp.pad(vals, ((0, 0), (0, _PAD - _DH)))
        tables.append(jnp.concatenate([vals, bases[h]], axis=0))
        wh = lax.dynamic_slice_in_dim(winner, h * _P, _P)
        idx2.append(jnp.where(wh >= 0, wh, _P + ar))
    new = _sc_gather2(tables[0], tables[1], idx2[0], idx2[1])

    y = _final(new, h0, h1, wo, bout.reshape(1, _EMB),
               w2[:, :_EMB], w2[:, _EMB:][:, sig2], b2)
    return y.reshape(1, _EMB, _IMG, _IMG)


# speedup vs baseline: 1.2050x; 1.2050x over previous
import functools

import jax
import jax.numpy as jnp
from jax import lax
from jax.experimental import pallas as pl
from jax.experimental.pallas import tpu as pltpu
from jax.experimental.pallas import tpu_sc as plsc

_HEADS = 2
_IMG = 224
_EMB = 192
_DH = _EMB // _HEADS
_CH = 384
_P = _IMG * _IMG
_EPS = 1e-5
_NUM_OBJ = _IMG // 2
_GROUPS = _HEADS * _IMG
_PAD = 128


def _conv1_body(x_ref, w_ref, b_ref, o0_ref, o1_ref):
    t = lax.dot_general(x_ref[...], w_ref[...], (((0,), (1,)), ((), ())),
                        preferred_element_type=jnp.float32)
    t = jnp.maximum(t + b_ref[...], 0.0)
    z = jnp.zeros((t.shape[0], _PAD - _DH), jnp.float32)
    o0_ref[...] = jnp.concatenate([t[:, :_DH], z], axis=1)
    o1_ref[...] = jnp.concatenate([t[:, _DH:], z], axis=1)


def _conv1(x2d, w, b, tp=1792):
    grid = _P // tp
    return pl.pallas_call(
        _conv1_body,
        grid=(grid,),
        in_specs=[
            pl.BlockSpec((_CH, tp), lambda i: (0, i)),
            pl.BlockSpec((_EMB, _CH), lambda i: (0, 0)),
            pl.BlockSpec((1, _EMB), lambda i: (0, 0)),
        ],
        out_specs=[
            pl.BlockSpec((tp, _PAD), lambda i: (i, 0)),
            pl.BlockSpec((tp, _PAD), lambda i: (i, 0)),
        ],
        out_shape=[
            jax.ShapeDtypeStruct((_P, _PAD), jnp.float32),
            jax.ShapeDtypeStruct((_P, _PAD), jnp.float32),
        ],
    )(x2d, w, b)


_AG = 8
_RUNS = [
    [(0, 0, 96), (1, 0, 96), (2, 0, 32)],
    [(2, 32, 64), (3, 0, 96), (4, 0, 64)],
    [(4, 64, 32), (5, 0, 96), (6, 0, 96)],
]


def _attn_body(g_ref, wq_ref, wk_ref, wv_ref, o_ref):
    for b in range(_AG):
        g = g_ref[b][:, :_DH]
        q = lax.dot_general(g, wq_ref[...], (((1,), (1,)), ((), ())),
                            preferred_element_type=jnp.float32)
        k = lax.dot_general(g, wk_ref[...], (((1,), (1,)), ((), ())),
                            preferred_element_type=jnp.float32)
        v = lax.dot_general(g, wv_ref[...], (((1,), (1,)), ((), ())),
                            preferred_element_type=jnp.float32)
        dots = lax.dot_general(q, k, (((1,), (1,)), ((), ())),
                               preferred_element_type=jnp.float32)
        dots = dots * (_DH ** -0.5)
        m = jnp.max(dots, axis=-1, keepdims=True)
        e = jnp.exp(dots - m)
        s = jnp.sum(e, axis=-1, keepdims=True)
        attn = e / s
        o = lax.dot_general(attn, v, (((1,), (0,)), ((), ())),
                            preferred_element_type=jnp.float32)
        o3 = o.reshape(32, 7, _DH)
        vals = None
        lane_a = lax.broadcasted_iota(jnp.int32, (32, _DH), 0)
        lane_c = lax.broadcasted_iota(jnp.int32, (32, _DH), 1)
        for t in range(3):
            parts = [lax.transpose(o3[:, qq, c0:c0 + ln], (1, 0))
                     for (qq, c0, ln) in _RUNS[t]]
            piece = jnp.concatenate(parts, axis=0)
            mt = (lane_c == 3 * lane_a + t).astype(jnp.float32)
            contrib = lax.dot_general(piece, mt, (((1,), (0,)), ((), ())),
                                      preferred_element_type=jnp.float32)
            vals = contrib if vals is None else vals + contrib
        o_ref[b] = vals


def _attention(g, wq, wk, wv):
    return pl.pallas_call(
        _attn_body,
        grid=(_GROUPS // _AG,),
        in_specs=[
            pl.BlockSpec((_AG, _IMG, _PAD), lambda i: (i, 0, 0)),
            pl.BlockSpec((_DH, _DH), lambda i: (0, 0)),
            pl.BlockSpec((_DH, _DH), lambda i: (0, 0)),
            pl.BlockSpec((_DH, _DH), lambda i: (0, 0)),
        ],
        out_specs=pl.BlockSpec((_AG, _IMG, _DH), lambda i: (i, 0, 0)),
        out_shape=jax.ShapeDtypeStruct((_GROUPS, _IMG, _DH), jnp.float32),
    )(g, wq, wk, wv)


def _final_body(n0_ref, n1_ref, h0_ref, h1_ref, wo_ref, bo_ref, w2a_ref,
                w2b_ref, b2_ref, o_ref):
    n = jnp.concatenate([n0_ref[:, :_DH], n1_ref[:, :_DH]], axis=1)
    a = lax.dot_general(n, wo_ref[...], (((1,), (1,)), ((), ())),
                        preferred_element_type=jnp.float32)
    a = jnp.maximum(a + bo_ref[...], 0.0)
    y = lax.dot_general(w2a_ref[...], a, (((1,), (1,)), ((), ())),
                        preferred_element_type=jnp.float32)
    hcat = jnp.concatenate([h0_ref[:, :_DH], h1_ref[:, :_DH]], axis=1)
    y = y + lax.dot_general(w2b_ref[...], hcat, (((1,), (1,)), ((), ())),
                            preferred_element_type=jnp.float32)
    o_ref[...] = jnp.maximum(y + b2_ref[...], 0.0)


def _final(new, h0, h1, wo, bo, w2a, w2b, b2, tp=1792):
    grid = _P // tp
    off = _P // tp
    return pl.pallas_call(
        _final_body,
        grid=(grid,),
        in_specs=[
            pl.BlockSpec((tp, _PAD), lambda i: (i, 0)),
            pl.BlockSpec((tp, _PAD), lambda i: (off + i, 0)),
            pl.BlockSpec((tp, _PAD), lambda i: (i, 0)),
            pl.BlockSpec((tp, _PAD), lambda i: (i, 0)),
            pl.BlockSpec((_EMB, _EMB), lambda i: (0, 0)),
            pl.BlockSpec((1, _EMB), lambda i: (0, 0)),
            pl.BlockSpec((_EMB, _EMB), lambda i: (0, 0)),
            pl.BlockSpec((_EMB, _EMB), lambda i: (0, 0)),
            pl.BlockSpec((_EMB, 1), lambda i: (0, 0)),
        ],
        out_specs=pl.BlockSpec((_EMB, tp), lambda i: (0, i)),
        out_shape=jax.ShapeDtypeStruct((_EMB, _P), jnp.float32),
    )(new, new, h0, h1, wo, bo, w2a, w2b, b2)


_NW = 32
_BPW = _P // _NW
_GC = 392


def _sc_gather2(t0, t1, i0, i1):
    mesh = plsc.VectorSubcoreMesh(core_axis_name="c", subcore_axis_name="s")

    @functools.partial(
        pl.kernel, mesh=mesh,
        out_type=jax.ShapeDtypeStruct((2 * _P, _PAD), jnp.float32),
        scratch_types=[
            pltpu.VMEM((_GC,), jnp.int32),
            pltpu.VMEM((_GC, _PAD), jnp.float32),
            pltpu.SemaphoreType.DMA,
        ],
    )
    def k(t0_hbm, t1_hbm, i0_hbm, i1_hbm, out_hbm, idx_v, rows_v, sem):
        wid = lax.axis_index("s") * 2 + lax.axis_index("c")
        for h in range(_HEADS):
            tab = t0_hbm if h == 0 else t1_hbm
            idx = i0_hbm if h == 0 else i1_hbm
            for c in range(_BPW // _GC):
                off = wid * _BPW + c * _GC
                pltpu.sync_copy(idx.at[pl.ds(off, _GC)], idx_v)
                pltpu.async_copy(tab.at[idx_v], rows_v, sem).wait()
                pltpu.sync_copy(rows_v, out_hbm.at[pl.ds(h * _P + off, _GC)])

    return k(t0, t1, i0, i1)


def _build_pix(prop, rand_inds):
    p = jnp.reshape(prop, (-1,))
    obj_mask = p > 0
    bg_mask = p <= 0
    obj_cnt = jnp.sum(obj_mask)
    bg_cnt = jnp.sum(bg_mask)
    obj_full = jnp.nonzero(obj_mask, size=_P, fill_value=0)[0]
    bg_full = jnp.nonzero(bg_mask, size=_P, fill_value=0)[0]
    ar = jnp.arange(_P)
    obj = jnp.where(obj_cnt == 0, ar, obj_full)
    bg = jnp.where(bg_cnt == 0, ar, bg_full)
    obj_sz = jnp.where(obj_cnt == 0, _P, obj_cnt)
    bg_sz = jnp.where(bg_cnt == 0, _P, bg_cnt)
    pix = []
    for h in range(_HEADS):
        op = obj[rand_inds[h, :_NUM_OBJ] % obj_sz]
        bp = bg[rand_inds[h, _NUM_OBJ:] % bg_sz]
        pix.append(jnp.concatenate([op, bp], axis=0).reshape(-1))
    return pix


def kernel(x, prop, conv1_w, bn1_gamma, bn1_beta, Wq, Wkv, Wout, bout,
           conv2_w, bn2_gamma, bn2_beta, rand_inds):
    inv = 1.0 / jnp.sqrt(jnp.float32(1.0 + _EPS))
    w1 = conv1_w.reshape(_EMB, _CH) * (bn1_gamma * inv)[:, None]
    b1 = bn1_beta.reshape(1, _EMB)
    w2 = conv2_w.reshape(_EMB, 2 * _EMB) * (bn2_gamma * inv)[:, None]
    b2 = bn2_beta.reshape(_EMB, 1)
    wk, wv = Wkv[:_DH], Wkv[_DH:]

    x2d = x.reshape(_CH, _P)
    h0, h1 = _conv1(x2d, w1, b1)
    bases = (h0, h1)

    pix = _build_pix(prop, rand_inds)

    g = _sc_gather2(h0, h1, pix[0], pix[1])
    g = g.reshape(_GROUPS, _IMG, _PAD)

    out = _attention(g, Wq, wk, wv)

    ar = jnp.arange(_P, dtype=jnp.int32)
    winner = jnp.full((2 * _P,), -1, jnp.int32).at[
        jnp.concatenate([pix[0], _P + pix[1]])].max(
        jnp.concatenate([ar, ar]))
    tables, idx2 = [], []
    for h in range(_HEADS):
        vals = out[h * _IMG:(h + 1) * _IMG].reshape(_P, _DH)
        vals = jnp.pad(vals, ((0, 0), (0, _PAD - _DH)))
        tables.append(jnp.concatenate([vals, bases[h]], axis=0))
        wh = lax.dynamic_slice_in_dim(winner, h * _P, _P)
        idx2.append(jnp.where(wh >= 0, wh, _P + ar))
    new = _sc_gather2(tables[0], tables[1], idx2[0], idx2[1])

    y = _final(new, h0, h1, Wout, bout.reshape(1, _EMB),
               w2[:, :_EMB], w2[:, _EMB:], b2)
    return y.reshape(1, _EMB, _IMG, _IMG)


# speedup vs baseline: 1.2942x; 1.0740x over previous
import functools

import jax
import jax.numpy as jnp
from jax import lax
from jax.experimental import pallas as pl
from jax.experimental.pallas import tpu as pltpu
from jax.experimental.pallas import tpu_sc as plsc

_HEADS = 2
_IMG = 224
_EMB = 192
_DH = _EMB // _HEADS
_CH = 384
_P = _IMG * _IMG
_EPS = 1e-5
_NUM_OBJ = _IMG // 2
_GROUPS = _HEADS * _IMG
_PAD = 128


def _conv1_body(x_ref, w_ref, b_ref, o0_ref, o1_ref):
    t = lax.dot_general(x_ref[...], w_ref[...], (((0,), (1,)), ((), ())),
                        preferred_element_type=jnp.float32)
    t = jnp.maximum(t + b_ref[...], 0.0)
    z = jnp.zeros((t.shape[0], _PAD - _DH), jnp.float32)
    o0_ref[...] = jnp.concatenate([t[:, :_DH], z], axis=1)
    o1_ref[...] = jnp.concatenate([t[:, _DH:], z], axis=1)


def _conv1(x2d, w, b, tp=1792):
    grid = _P // tp
    return pl.pallas_call(
        _conv1_body,
        grid=(grid,),
        in_specs=[
            pl.BlockSpec((_CH, tp), lambda i: (0, i)),
            pl.BlockSpec((_EMB, _CH), lambda i: (0, 0)),
            pl.BlockSpec((1, _EMB), lambda i: (0, 0)),
        ],
        out_specs=[
            pl.BlockSpec((tp, _PAD), lambda i: (i, 0)),
            pl.BlockSpec((tp, _PAD), lambda i: (i, 0)),
        ],
        out_shape=[
            jax.ShapeDtypeStruct((_P, _PAD), jnp.float32),
            jax.ShapeDtypeStruct((_P, _PAD), jnp.float32),
        ],
    )(x2d, w, b)


_AG = 8
_RUNS = [
    [(0, 0, 96), (1, 0, 96), (2, 0, 32)],
    [(2, 32, 64), (3, 0, 96), (4, 0, 64)],
    [(4, 64, 32), (5, 0, 96), (6, 0, 96)],
]


def _attn_body(g_ref, wq_ref, wk_ref, wv_ref, o_ref):
    for b in range(_AG):
        g = g_ref[b][:, :_DH]
        q = lax.dot_general(g, wq_ref[...], (((1,), (1,)), ((), ())),
                            preferred_element_type=jnp.float32)
        k = lax.dot_general(g, wk_ref[...], (((1,), (1,)), ((), ())),
                            preferred_element_type=jnp.float32)
        v = lax.dot_general(g, wv_ref[...], (((1,), (1,)), ((), ())),
                            preferred_element_type=jnp.float32)
        dots = lax.dot_general(q, k, (((1,), (1,)), ((), ())),
                               preferred_element_type=jnp.float32)
        dots = dots * (_DH ** -0.5)
        m = jnp.max(dots, axis=-1, keepdims=True)
        e = jnp.exp(dots - m)
        s = jnp.sum(e, axis=-1, keepdims=True)
        attn = e / s
        o = lax.dot_general(attn, v, (((1,), (0,)), ((), ())),
                            preferred_element_type=jnp.float32)
        o3 = o.reshape(32, 7, _DH)
        vals = None
        lane_a = lax.broadcasted_iota(jnp.int32, (32, _DH), 0)
        lane_c = lax.broadcasted_iota(jnp.int32, (32, _DH), 1)
        for t in range(3):
            parts = [lax.transpose(o3[:, qq, c0:c0 + ln], (1, 0))
                     for (qq, c0, ln) in _RUNS[t]]
            piece = jnp.concatenate(parts, axis=0)
            mt = (lane_c == 3 * lane_a + t).astype(jnp.float32)
            contrib = lax.dot_general(piece, mt, (((1,), (0,)), ((), ())),
                                      preferred_element_type=jnp.float32)
            vals = contrib if vals is None else vals + contrib
        o_ref[b] = vals


def _attention(g, wq, wk, wv):
    return pl.pallas_call(
        _attn_body,
        grid=(_GROUPS // _AG,),
        in_specs=[
            pl.BlockSpec((_AG, _IMG, _PAD), lambda i: (i, 0, 0)),
            pl.BlockSpec((_DH, _DH), lambda i: (0, 0)),
            pl.BlockSpec((_DH, _DH), lambda i: (0, 0)),
            pl.BlockSpec((_DH, _DH), lambda i: (0, 0)),
        ],
        out_specs=pl.BlockSpec((_AG, _IMG, _DH), lambda i: (i, 0, 0)),
        out_shape=jax.ShapeDtypeStruct((_GROUPS, _IMG, _DH), jnp.float32),
    )(g, wq, wk, wv)


def _final_body(n0_ref, n1_ref, h0_ref, h1_ref, wo_ref, bo_ref, w2a_ref,
                w2b_ref, b2_ref, o_ref):
    n = jnp.concatenate([n0_ref[:, :_DH], n1_ref[:, :_DH]], axis=1)
    a = lax.dot_general(n, wo_ref[...], (((1,), (1,)), ((), ())),
                        preferred_element_type=jnp.float32)
    a = jnp.maximum(a + bo_ref[...], 0.0)
    y = lax.dot_general(w2a_ref[...], a, (((1,), (1,)), ((), ())),
                        preferred_element_type=jnp.float32)
    hcat = jnp.concatenate([h0_ref[:, :_DH], h1_ref[:, :_DH]], axis=1)
    y = y + lax.dot_general(w2b_ref[...], hcat, (((1,), (1,)), ((), ())),
                            preferred_element_type=jnp.float32)
    o_ref[...] = jnp.maximum(y + b2_ref[...], 0.0)


def _final(new, h0, h1, wo, bo, w2a, w2b, b2, tp=1792):
    grid = _P // tp
    off = _P // tp
    return pl.pallas_call(
        _final_body,
        grid=(grid,),
        in_specs=[
            pl.BlockSpec((tp, _PAD), lambda i: (i, 0)),
            pl.BlockSpec((tp, _PAD), lambda i: (off + i, 0)),
            pl.BlockSpec((tp, _PAD), lambda i: (i, 0)),
            pl.BlockSpec((tp, _PAD), lambda i: (i, 0)),
            pl.BlockSpec((_EMB, _EMB), lambda i: (0, 0)),
            pl.BlockSpec((1, _EMB), lambda i: (0, 0)),
            pl.BlockSpec((_EMB, _EMB), lambda i: (0, 0)),
            pl.BlockSpec((_EMB, _EMB), lambda i: (0, 0)),
            pl.BlockSpec((_EMB, 1), lambda i: (0, 0)),
        ],
        out_specs=pl.BlockSpec((_EMB, tp), lambda i: (0, i)),
        out_shape=jax.ShapeDtypeStruct((_EMB, _P), jnp.float32),
    )(new, new, h0, h1, wo, bo, w2a, w2b, b2)


_NW = 32
_BPW = _P // _NW
_GC = 392


def _sc_gather2(t0, t1, i0, i1):
    mesh = plsc.VectorSubcoreMesh(core_axis_name="c", subcore_axis_name="s")

    @functools.partial(
        pl.kernel, mesh=mesh,
        out_type=jax.ShapeDtypeStruct((2 * _P, _PAD), jnp.float32),
        scratch_types=[
            pltpu.VMEM((_GC,), jnp.int32),
            pltpu.VMEM((_GC, _PAD), jnp.float32),
            pltpu.SemaphoreType.DMA,
        ],
    )
    def k(t0_hbm, t1_hbm, i0_hbm, i1_hbm, out_hbm, idx_v, rows_v, sem):
        wid = lax.axis_index("s") * 2 + lax.axis_index("c")
        for h in range(_HEADS):
            tab = t0_hbm if h == 0 else t1_hbm
            idx = i0_hbm if h == 0 else i1_hbm
            for c in range(_BPW // _GC):
                off = wid * _BPW + c * _GC
                pltpu.sync_copy(idx.at[pl.ds(off, _GC)], idx_v)
                pltpu.async_copy(tab.at[idx_v], rows_v, sem).wait()
                pltpu.sync_copy(rows_v, out_hbm.at[pl.ds(h * _P + off, _GC)])

    return k(t0, t1, i0, i1)


def _build_pix(prop, rand_inds):
    p = jnp.reshape(prop, (-1,))
    obj_mask = p > 0
    bg_mask = p <= 0
    obj_cnt = jnp.sum(obj_mask)
    bg_cnt = jnp.sum(bg_mask)
    obj_full = jnp.nonzero(obj_mask, size=_P, fill_value=0)[0]
    bg_full = jnp.nonzero(bg_mask, size=_P, fill_value=0)[0]
    ar = jnp.arange(_P)
    obj = jnp.where(obj_cnt == 0, ar, obj_full)
    bg = jnp.where(bg_cnt == 0, ar, bg_full)
    obj_sz = jnp.where(obj_cnt == 0, _P, obj_cnt)
    bg_sz = jnp.where(bg_cnt == 0, _P, bg_cnt)
    pix = []
    for h in range(_HEADS):
        op = obj[rand_inds[h, :_NUM_OBJ] % obj_sz]
        bp = bg[rand_inds[h, _NUM_OBJ:] % bg_sz]
        pix.append(jnp.concatenate([op, bp], axis=0).reshape(-1))
    return pix


def kernel(x, prop, conv1_w, bn1_gamma, bn1_beta, Wq, Wkv, Wout, bout,
           conv2_w, bn2_gamma, bn2_beta, rand_inds):
    inv = 1.0 / jnp.sqrt(jnp.float32(1.0 + _EPS))
    w1 = conv1_w.reshape(_EMB, _CH) * (bn1_gamma * inv)[:, None]
    b1 = bn1_beta.reshape(1, _EMB)
    w2 = conv2_w.reshape(_EMB, 2 * _EMB) * (bn2_gamma * inv)[:, None]
    b2 = bn2_beta.reshape(_EMB, 1)
    wk, wv = Wkv[:_DH], Wkv[_DH:]

    x2d = x.reshape(_CH, _P)
    h0, h1 = _conv1(x2d, w1, b1)
    bases = (h0, h1)

    pix = _build_pix(prop, rand_inds)

    g = _sc_gather2(h0, h1, pix[0], pix[1])
    g = g.reshape(_GROUPS, _IMG, _PAD)

    out = _attention(g, Wq, wk, wv)

    ar = jnp.arange(_P, dtype=jnp.int32)
    tables, idx2 = [], []
    for h in range(_HEADS):
        vals = out[h * _IMG:(h + 1) * _IMG].reshape(_P, _DH)
        vals = jnp.pad(vals, ((0, 0), (0, _PAD - _DH)))
        tables.append(jnp.concatenate([vals, bases[h]], axis=0))
        wh = jnp.full((_P,), -1, jnp.int32).at[pix[h]].max(ar)
        idx2.append(jnp.where(wh >= 0, wh, _P + ar))
    new = _sc_gather2(tables[0], tables[1], idx2[0], idx2[1])

    y = _final(new, h0, h1, Wout, bout.reshape(1, _EMB),
               w2[:, :_EMB], w2[:, _EMB:], b2)
    return y.reshape(1, _EMB, _IMG, _IMG)


# speedup vs baseline: 1.3131x; 1.0146x over previous
import functools

import jax
import jax.numpy as jnp
from jax import lax
from jax.experimental import pallas as pl
from jax.experimental.pallas import tpu as pltpu
from jax.experimental.pallas import tpu_sc as plsc

_HEADS = 2
_IMG = 224
_EMB = 192
_DH = _EMB // _HEADS
_CH = 384
_P = _IMG * _IMG
_EPS = 1e-5
_NUM_OBJ = _IMG // 2
_GROUPS = _HEADS * _IMG
_PAD = 128


def _conv1_body(x_ref, w_ref, b_ref, o0_ref, o1_ref):
    t = lax.dot_general(x_ref[...], w_ref[...], (((0,), (1,)), ((), ())),
                        preferred_element_type=jnp.float32)
    t = jnp.maximum(t + b_ref[...], 0.0)
    z = jnp.zeros((t.shape[0], _PAD - _DH), jnp.float32)
    o0_ref[...] = jnp.concatenate([t[:, :_DH], z], axis=1)
    o1_ref[...] = jnp.concatenate([t[:, _DH:], z], axis=1)


def _conv1(x2d, w, b, tp=3584):
    grid = _P // tp
    return pl.pallas_call(
        _conv1_body,
        grid=(grid,),
        in_specs=[
            pl.BlockSpec((_CH, tp), lambda i: (0, i)),
            pl.BlockSpec((_EMB, _CH), lambda i: (0, 0)),
            pl.BlockSpec((1, _EMB), lambda i: (0, 0)),
        ],
        out_specs=[
            pl.BlockSpec((tp, _PAD), lambda i: (i, 0)),
            pl.BlockSpec((tp, _PAD), lambda i: (i, 0)),
        ],
        out_shape=[
            jax.ShapeDtypeStruct((_P, _PAD), jnp.float32),
            jax.ShapeDtypeStruct((_P, _PAD), jnp.float32),
        ],
    )(x2d, w, b)


_AG = 32
_RUNS = [
    [(0, 0, 96), (1, 0, 96), (2, 0, 32)],
    [(2, 32, 64), (3, 0, 96), (4, 0, 64)],
    [(4, 64, 32), (5, 0, 96), (6, 0, 96)],
]


def _attn_body(g_ref, wq_ref, wk_ref, wv_ref, o_ref):
    for b in range(_AG):
        g = g_ref[b][:, :_DH]
        q = lax.dot_general(g, wq_ref[...], (((1,), (1,)), ((), ())),
                            preferred_element_type=jnp.float32)
        k = lax.dot_general(g, wk_ref[...], (((1,), (1,)), ((), ())),
                            preferred_element_type=jnp.float32)
        v = lax.dot_general(g, wv_ref[...], (((1,), (1,)), ((), ())),
                            preferred_element_type=jnp.float32)
        dots = lax.dot_general(q, k, (((1,), (1,)), ((), ())),
                               preferred_element_type=jnp.float32)
        dots = dots * (_DH ** -0.5)
        m = jnp.max(dots, axis=-1, keepdims=True)
        e = jnp.exp(dots - m)
        s = jnp.sum(e, axis=-1, keepdims=True)
        attn = e / s
        o = lax.dot_general(attn, v, (((1,), (0,)), ((), ())),
                            preferred_element_type=jnp.float32)
        o3 = o.reshape(32, 7, _DH)
        vals = None
        lane_a = lax.broadcasted_iota(jnp.int32, (32, _DH), 0)
        lane_c = lax.broadcasted_iota(jnp.int32, (32, _DH), 1)
        for t in range(3):
            parts = [lax.transpose(o3[:, qq, c0:c0 + ln], (1, 0))
                     for (qq, c0, ln) in _RUNS[t]]
            piece = jnp.concatenate(parts, axis=0)
            mt = (lane_c == 3 * lane_a + t).astype(jnp.float32)
            contrib = lax.dot_general(piece, mt, (((1,), (0,)), ((), ())),
                                      preferred_element_type=jnp.float32)
            vals = contrib if vals is None else vals + contrib
        o_ref[b] = vals


def _attention(g, wq, wk, wv):
    return pl.pallas_call(
        _attn_body,
        grid=(_GROUPS // _AG,),
        in_specs=[
            pl.BlockSpec((_AG, _IMG, _PAD), lambda i: (i, 0, 0)),
            pl.BlockSpec((_DH, _DH), lambda i: (0, 0)),
            pl.BlockSpec((_DH, _DH), lambda i: (0, 0)),
            pl.BlockSpec((_DH, _DH), lambda i: (0, 0)),
        ],
        out_specs=pl.BlockSpec((_AG, _IMG, _DH), lambda i: (i, 0, 0)),
        out_shape=jax.ShapeDtypeStruct((_GROUPS, _IMG, _DH), jnp.float32),
    )(g, wq, wk, wv)


def _final_body(n0_ref, n1_ref, h0_ref, h1_ref, wo_ref, bo_ref, w2a_ref,
                w2b_ref, b2_ref, o_ref):
    n = jnp.concatenate([n0_ref[:, :_DH], n1_ref[:, :_DH]], axis=1)
    a = lax.dot_general(n, wo_ref[...], (((1,), (1,)), ((), ())),
                        preferred_element_type=jnp.float32)
    a = jnp.maximum(a + bo_ref[...], 0.0)
    y = lax.dot_general(w2a_ref[...], a, (((1,), (1,)), ((), ())),
                        preferred_element_type=jnp.float32)
    hcat = jnp.concatenate([h0_ref[:, :_DH], h1_ref[:, :_DH]], axis=1)
    y = y + lax.dot_general(w2b_ref[...], hcat, (((1,), (1,)), ((), ())),
                            preferred_element_type=jnp.float32)
    o_ref[...] = jnp.maximum(y + b2_ref[...], 0.0)


def _final(new, h0, h1, wo, bo, w2a, w2b, b2, tp=3584):
    grid = _P // tp
    off = _P // tp
    return pl.pallas_call(
        _final_body,
        grid=(grid,),
        in_specs=[
            pl.BlockSpec((tp, _PAD), lambda i: (i, 0)),
            pl.BlockSpec((tp, _PAD), lambda i: (off + i, 0)),
            pl.BlockSpec((tp, _PAD), lambda i: (i, 0)),
            pl.BlockSpec((tp, _PAD), lambda i: (i, 0)),
            pl.BlockSpec((_EMB, _EMB), lambda i: (0, 0)),
            pl.BlockSpec((1, _EMB), lambda i: (0, 0)),
            pl.BlockSpec((_EMB, _EMB), lambda i: (0, 0)),
            pl.BlockSpec((_EMB, _EMB), lambda i: (0, 0)),
            pl.BlockSpec((_EMB, 1), lambda i: (0, 0)),
        ],
        out_specs=pl.BlockSpec((_EMB, tp), lambda i: (0, i)),
        out_shape=jax.ShapeDtypeStruct((_EMB, _P), jnp.float32),
    )(new, new, h0, h1, wo, bo, w2a, w2b, b2)


_NW = 32
_BPW = _P // _NW
_GC = 392


def _sc_gather2(t0, t1, i0, i1):
    mesh = plsc.VectorSubcoreMesh(core_axis_name="c", subcore_axis_name="s")

    @functools.partial(
        pl.kernel, mesh=mesh,
        out_type=jax.ShapeDtypeStruct((2 * _P, _PAD), jnp.float32),
        scratch_types=[
            pltpu.VMEM((_GC,), jnp.int32),
            pltpu.VMEM((_GC, _PAD), jnp.float32),
            pltpu.SemaphoreType.DMA,
        ],
    )
    def k(t0_hbm, t1_hbm, i0_hbm, i1_hbm, out_hbm, idx_v, rows_v, sem):
        wid = lax.axis_index("s") * 2 + lax.axis_index("c")
        for h in range(_HEADS):
            tab = t0_hbm if h == 0 else t1_hbm
            idx = i0_hbm if h == 0 else i1_hbm
            for c in range(_BPW // _GC):
                off = wid * _BPW + c * _GC
                pltpu.sync_copy(idx.at[pl.ds(off, _GC)], idx_v)
                pltpu.async_copy(tab.at[idx_v], rows_v, sem).wait()
                pltpu.sync_copy(rows_v, out_hbm.at[pl.ds(h * _P + off, _GC)])

    return k(t0, t1, i0, i1)


def _build_pix(prop, rand_inds):
    p = jnp.reshape(prop, (-1,))
    obj_mask = p > 0
    bg_mask = p <= 0
    obj_cnt = jnp.sum(obj_mask)
    bg_cnt = jnp.sum(bg_mask)
    obj_full = jnp.nonzero(obj_mask, size=_P, fill_value=0)[0]
    bg_full = jnp.nonzero(bg_mask, size=_P, fill_value=0)[0]
    ar = jnp.arange(_P)
    obj = jnp.where(obj_cnt == 0, ar, obj_full)
    bg = jnp.where(bg_cnt == 0, ar, bg_full)
    obj_sz = jnp.where(obj_cnt == 0, _P, obj_cnt)
    bg_sz = jnp.where(bg_cnt == 0, _P, bg_cnt)
    pix = []
    for h in range(_HEADS):
        op = obj[rand_inds[h, :_NUM_OBJ] % obj_sz]
        bp = bg[rand_inds[h, _NUM_OBJ:] % bg_sz]
        pix.append(jnp.concatenate([op, bp], axis=0).reshape(-1))
    return pix


def kernel(x, prop, conv1_w, bn1_gamma, bn1_beta, Wq, Wkv, Wout, bout,
           conv2_w, bn2_gamma, bn2_beta, rand_inds):
    inv = 1.0 / jnp.sqrt(jnp.float32(1.0 + _EPS))
    w1 = conv1_w.reshape(_EMB, _CH) * (bn1_gamma * inv)[:, None]
    b1 = bn1_beta.reshape(1, _EMB)
    w2 = conv2_w.reshape(_EMB, 2 * _EMB) * (bn2_gamma * inv)[:, None]
    b2 = bn2_beta.reshape(_EMB, 1)
    wk, wv = Wkv[:_DH], Wkv[_DH:]

    x2d = x.reshape(_CH, _P)
    h0, h1 = _conv1(x2d, w1, b1)
    bases = (h0, h1)

    pix = _build_pix(prop, rand_inds)

    g = _sc_gather2(h0, h1, pix[0], pix[1])
    g = g.reshape(_GROUPS, _IMG, _PAD)

    out = _attention(g, Wq, wk, wv)

    ar = jnp.arange(_P, dtype=jnp.int32)
    tables, idx2 = [], []
    for h in range(_HEADS):
        vals = out[h * _IMG:(h + 1) * _IMG].reshape(_P, _DH)
        vals = jnp.pad(vals, ((0, 0), (0, _PAD - _DH)))
        tables.append(jnp.concatenate([vals, bases[h]], axis=0))
        wh = jnp.full((_P,), -1, jnp.int32).at[pix[h]].max(ar)
        idx2.append(jnp.where(wh >= 0, wh, _P + ar))
    new = _sc_gather2(tables[0], tables[1], idx2[0], idx2[1])

    y = _final(new, h0, h1, Wout, bout.reshape(1, _EMB),
               w2[:, :_EMB], w2[:, _EMB:], b2)
    return y.reshape(1, _EMB, _IMG, _IMG)
